# baseline (speedup 1.0000x reference)
import functools
import math

import jax
import jax.numpy as jnp
from jax import lax
from jax.experimental import pallas as pl
from jax.experimental.pallas import tpu as pltpu
from jax.experimental.pallas import tpu_sc as plsc

NC = 2
NS = 16
N_TILES = NC * NS
STRIP = 128
CHUNK = 64


def _round_up(x, m):
    return (x + m - 1) // m * m


def _sc_degree_partials(src_pad, dst_pad, zeros2):
    epad = src_pad.shape[0]
    npad = zeros2.shape[1]
    eper = epad // N_TILES
    mesh = plsc.VectorSubcoreMesh(core_axis_name="c", subcore_axis_name="s")

    @functools.partial(
        pl.kernel,
        out_type=jax.ShapeDtypeStruct((N_TILES, 2, npad), jnp.float32),
        mesh=mesh,
        scratch_types=[
            pltpu.VMEM((eper,), jnp.int32),
            pltpu.VMEM((eper,), jnp.int32),
            pltpu.VMEM((npad,), jnp.float32),
            pltpu.VMEM((npad,), jnp.float32),
        ],
        compiler_params=pltpu.CompilerParams(needs_layout_passes=False),
    )
    def deg_kernel(src_hbm, dst_hbm, z_hbm, out_hbm, src_v, dst_v, hs_v, hd_v):
        cid = lax.axis_index("c")
        sid = lax.axis_index("s")
        wid = cid * NS + sid
        base = wid * eper
        pltpu.sync_copy(z_hbm.at[0], hs_v)
        pltpu.sync_copy(z_hbm.at[1], hd_v)
        pltpu.sync_copy(src_hbm.at[pl.ds(base, eper)], src_v)
        pltpu.sync_copy(dst_hbm.at[pl.ds(base, eper)], dst_v)
        ones = jnp.full((16,), 1.0, jnp.float32)

        def body(g, carry):
            vs = src_v[pl.ds(g * 16, 16)]
            plsc.addupdate_scatter(hs_v, [vs], ones)
            vd = dst_v[pl.ds(g * 16, 16)]
            plsc.addupdate_scatter(hd_v, [vd], ones)
            return carry

        lax.fori_loop(0, eper // 16, body, 0)
        pltpu.sync_copy(hs_v, out_hbm.at[wid, 0])
        pltpu.sync_copy(hd_v, out_hbm.at[wid, 1])

    return deg_kernel(src_pad, dst_pad, zeros2)


def _tc_norms(partials):
    t, two, npad = partials.shape

    def body(p_ref, out_ref):
        x = p_ref[...].reshape(t, two * npad)
        deg = jnp.sum(x, axis=0)
        out_ref[...] = lax.rsqrt(jnp.maximum(deg, 1.0)).reshape(two, npad)

    return pl.pallas_call(
        body,
        out_shape=jax.ShapeDtypeStruct((two, npad), jnp.float32),
    )(partials)


def _sc_aggregate(ht_flat, srcp, dst2, zeros_mat):
    n_strips = srcp.shape[0]
    nch2 = srcp.shape[3]
    npad = zeros_mat.shape[0]
    rpt = npad // NS
    strips_per_core = n_strips // NC
    mesh = plsc.VectorSubcoreMesh(core_axis_name="c", subcore_axis_name="s")

    @functools.partial(
        pl.kernel,
        out_type=jax.ShapeDtypeStruct((n_strips * npad, STRIP), jnp.float32),
        mesh=mesh,
        scratch_types=[
            pltpu.VMEM((nch2, CHUNK), jnp.int32),
            pltpu.VMEM((nch2, CHUNK), jnp.int32),
            pltpu.VMEM((CHUNK, STRIP), jnp.float32),
            pltpu.VMEM((CHUNK, STRIP), jnp.float32),
            pltpu.VMEM_SHARED((npad, STRIP), jnp.float32),
            pltpu.SemaphoreType.DMA,
            pltpu.SemaphoreType.DMA,
        ],
    )
    def agg_kernel(ht_hbm, src_hbm, dst_hbm, z_hbm, out_hbm,
                   src_v, dst_v, r0, r1, acc, sem0, sem1):
        cid = lax.axis_index("c")
        sid = lax.axis_index("s")
        for j in range(strips_per_core):
            s = cid * strips_per_core + j
            pltpu.sync_copy(z_hbm.at[pl.ds(sid * rpt, rpt)],
                            acc.at[pl.ds(sid * rpt, rpt)])
            plsc.subcore_barrier()
            for half in range(2):
                pltpu.sync_copy(src_hbm.at[s, sid, half], src_v)
                pltpu.sync_copy(dst_hbm.at[sid, half], dst_v)
                pltpu.async_copy(ht_hbm.at[src_v.at[0]], r0, sem0)

                def body(i, carry):
                    c0 = 2 * i
                    pltpu.make_async_copy(
                        ht_hbm.at[src_v.at[c0]], r0, sem0).wait()
                    pltpu.async_copy(ht_hbm.at[src_v.at[c0 + 1]], r1, sem1)
                    pltpu.sync_copy(r0, acc.at[dst_v.at[c0]], add=True)
                    pltpu.make_async_copy(
                        ht_hbm.at[src_v.at[c0 + 1]], r1, sem1).wait()
                    nxt = jnp.minimum(c0 + 2, nch2 - 1)
                    pltpu.async_copy(ht_hbm.at[src_v.at[nxt]], r0, sem0)
                    pltpu.sync_copy(r1, acc.at[dst_v.at[c0 + 1]], add=True)
                    return carry

                lax.fori_loop(0, nch2 // 2, body, 0)
                pltpu.make_async_copy(
                    ht_hbm.at[src_v.at[nch2 - 1]], r0, sem0).wait()
            plsc.subcore_barrier()
            pltpu.sync_copy(
                acc.at[pl.ds(sid * rpt, rpt)],
                out_hbm.at[pl.ds(s * npad + sid * rpt, rpt)])
            plsc.subcore_barrier()

    return agg_kernel(ht_flat, srcp, dst2, zeros_mat)


def _tc_pre(h, ns_mat, w_l):
    npad, d = h.shape
    n_strips = d // STRIP
    br = 512

    def body(h_ref, ns_ref, w_ref, out_ref):
        x = h_ref[...] * ns_ref[:, :1]
        y = jnp.dot(x, w_ref[...], preferred_element_type=jnp.float32)
        for s in range(n_strips):
            out_ref[s] = y[:, s * STRIP:(s + 1) * STRIP]

    return pl.pallas_call(
        body,
        grid=(npad // br,),
        in_specs=[
            pl.BlockSpec((br, d), lambda i: (i, 0)),
            pl.BlockSpec((br, STRIP), lambda i: (i, 0)),
            pl.BlockSpec((d, d), lambda i: (0, 0)),
        ],
        out_specs=pl.BlockSpec((n_strips, br, STRIP), lambda i: (0, i, 0)),
        out_shape=jax.ShapeDtypeStruct((n_strips, npad, STRIP), jnp.float32),
    )(h, ns_mat, w_l)


def _tc_post(agg, nd_mat, b_l, g_l, bt_l):
    n_strips, npad, strip = agg.shape
    d = n_strips * strip
    br = 512
    inv_sqrt2 = 1.0 / math.sqrt(2.0)

    def body(a_ref, nd_ref, b_ref, g_ref, bt_ref, out_ref):
        x = jnp.concatenate([a_ref[s] for s in range(n_strips)], axis=1)
        x = x * nd_ref[:, :1] + b_ref[...]
        mu = jnp.mean(x, axis=-1, keepdims=True)
        xc = x - mu
        var = jnp.mean(xc * xc, axis=-1, keepdims=True)
        xn = xc * lax.rsqrt(var + 1e-5) * g_ref[...] + bt_ref[...]
        out_ref[...] = 0.5 * xn * (1.0 + lax.erf(xn * inv_sqrt2))

    return pl.pallas_call(
        body,
        grid=(npad // br,),
        in_specs=[
            pl.BlockSpec((n_strips, br, strip), lambda i: (0, i, 0)),
            pl.BlockSpec((br, STRIP), lambda i: (i, 0)),
            pl.BlockSpec((1, d), lambda i: (0, 0)),
            pl.BlockSpec((1, d), lambda i: (0, 0)),
            pl.BlockSpec((1, d), lambda i: (0, 0)),
        ],
        out_specs=pl.BlockSpec((br, d), lambda i: (i, 0)),
        out_shape=jax.ShapeDtypeStruct((npad, d), jnp.float32),
    )(agg, nd_mat, b_l, g_l, bt_l)


def kernel(features, edge_index, W, b, gamma, beta):
    n, d = features.shape
    e = edge_index.shape[1]
    n_layers = W.shape[0]
    n_strips = d // STRIP

    npad = _round_up(n, 2048)
    epad = _round_up(e, NS * CHUNK * 4)
    sink = npad - 1

    src = edge_index[0].astype(jnp.int32)
    dst = edge_index[1].astype(jnp.int32)
    src_pad = jnp.concatenate(
        [src, jnp.full((epad - e,), sink, jnp.int32)])
    dst_pad = jnp.concatenate(
        [dst, jnp.full((epad - e,), sink, jnp.int32)])

    srcp = (src_pad[None, :]
            + (jnp.arange(n_strips, dtype=jnp.int32) * npad)[:, None])
    srcp = srcp.reshape(n_strips, NS, 2, epad // NS // CHUNK // 2, CHUNK)
    dst2 = dst_pad.reshape(NS, 2, epad // NS // CHUNK // 2, CHUNK)

    zeros2 = jnp.zeros((2, npad), jnp.float32)
    zeros_mat = jnp.zeros((npad, STRIP), jnp.float32)

    partials = _sc_degree_partials(src_pad, dst_pad, zeros2)
    norms = _tc_norms(partials)
    ns_mat = jnp.broadcast_to(norms[0][:, None], (npad, STRIP))
    nd_mat = jnp.broadcast_to(norms[1][:, None], (npad, STRIP))

    h = jnp.pad(features, ((0, npad - n), (0, 0)))
    for l in range(n_layers):
        ht = _tc_pre(h, ns_mat, W[l])
        agg = _sc_aggregate(ht.reshape(n_strips * npad, STRIP),
                            srcp, dst2, zeros_mat)
        h = _tc_post(agg.reshape(n_strips, npad, STRIP), nd_mat,
                     b[l].reshape(1, d), gamma[l].reshape(1, d),
                     beta[l].reshape(1, d))
    return h[:n]


# speedup vs baseline: 1.1039x; 1.1039x over previous
import functools
import math

import jax
import jax.numpy as jnp
from jax import lax
from jax.experimental import pallas as pl
from jax.experimental.pallas import tpu as pltpu
from jax.experimental.pallas import tpu_sc as plsc

NC = 2
NS = 16
N_TILES = NC * NS
STRIP = 128
CHUNK = 128
QTRS = 4


def _round_up(x, m):
    return (x + m - 1) // m * m


def _sc_degree_partials(src_pad, dst_pad, zeros2):
    epad = src_pad.shape[0]
    npad = zeros2.shape[1]
    eper = epad // N_TILES
    mesh = plsc.VectorSubcoreMesh(core_axis_name="c", subcore_axis_name="s")

    @functools.partial(
        pl.kernel,
        out_type=jax.ShapeDtypeStruct((N_TILES, 2, npad), jnp.float32),
        mesh=mesh,
        scratch_types=[
            pltpu.VMEM((eper,), jnp.int32),
            pltpu.VMEM((eper,), jnp.int32),
            pltpu.VMEM((npad,), jnp.float32),
            pltpu.VMEM((npad,), jnp.float32),
        ],
        compiler_params=pltpu.CompilerParams(needs_layout_passes=False),
    )
    def deg_kernel(src_hbm, dst_hbm, z_hbm, out_hbm, src_v, dst_v, hs_v, hd_v):
        cid = lax.axis_index("c")
        sid = lax.axis_index("s")
        wid = cid * NS + sid
        base = wid * eper
        pltpu.sync_copy(z_hbm.at[0], hs_v)
        pltpu.sync_copy(z_hbm.at[1], hd_v)
        pltpu.sync_copy(src_hbm.at[pl.ds(base, eper)], src_v)
        pltpu.sync_copy(dst_hbm.at[pl.ds(base, eper)], dst_v)
        ones = jnp.full((16,), 1.0, jnp.float32)

        def body(g, carry):
            vs = src_v[pl.ds(g * 16, 16)]
            plsc.addupdate_scatter(hs_v, [vs], ones)
            vd = dst_v[pl.ds(g * 16, 16)]
            plsc.addupdate_scatter(hd_v, [vd], ones)
            return carry

        lax.fori_loop(0, eper // 16, body, 0)
        pltpu.sync_copy(hs_v, out_hbm.at[wid, 0])
        pltpu.sync_copy(hd_v, out_hbm.at[wid, 1])

    return deg_kernel(src_pad, dst_pad, zeros2)


def _tc_norms(partials):
    t, two, npad = partials.shape

    def body(p_ref, out_ref):
        x = p_ref[...].reshape(t, two * npad)
        deg = jnp.sum(x, axis=0)
        out_ref[...] = lax.rsqrt(jnp.maximum(deg, 1.0)).reshape(two, npad)

    return pl.pallas_call(
        body,
        out_shape=jax.ShapeDtypeStruct((two, npad), jnp.float32),
    )(partials)


def _sc_aggregate(ht_flat, srcp, dst2, zeros_mat):
    n_strips = srcp.shape[0]
    nq = srcp.shape[3]
    npad = zeros_mat.shape[0]
    rpt = npad // NS
    strips_per_core = n_strips // NC
    mesh = plsc.VectorSubcoreMesh(core_axis_name="c", subcore_axis_name="s")

    @functools.partial(
        pl.kernel,
        out_type=jax.ShapeDtypeStruct((n_strips * npad, STRIP), jnp.float32),
        mesh=mesh,
        scratch_types=[
            pltpu.VMEM((nq, CHUNK), jnp.int32),
            pltpu.VMEM((nq, CHUNK), jnp.int32),
            pltpu.VMEM((CHUNK, STRIP), jnp.float32),
            pltpu.VMEM((CHUNK, STRIP), jnp.float32),
            pltpu.VMEM_SHARED((npad, STRIP), jnp.float32),
            pltpu.SemaphoreType.DMA,
            pltpu.SemaphoreType.DMA,
        ],
    )
    def agg_kernel(ht_hbm, src_hbm, dst_hbm, z_hbm, out_hbm,
                   src_v, dst_v, r0, r1, acc, sem0, sem1):
        cid = lax.axis_index("c")
        sid = lax.axis_index("s")
        for j in range(strips_per_core):
            s = cid * strips_per_core + j
            pltpu.sync_copy(z_hbm.at[pl.ds(sid * rpt, rpt)],
                            acc.at[pl.ds(sid * rpt, rpt)])
            plsc.subcore_barrier()
            for q in range(QTRS):
                pltpu.sync_copy(src_hbm.at[s, sid, q], src_v)
                pltpu.sync_copy(dst_hbm.at[sid, q], dst_v)
                pltpu.async_copy(ht_hbm.at[src_v.at[0]], r0, sem0)

                def body(i, carry):
                    c0 = 2 * i
                    pltpu.make_async_copy(
                        ht_hbm.at[src_v.at[c0]], r0, sem0).wait()
                    pltpu.async_copy(ht_hbm.at[src_v.at[c0 + 1]], r1, sem1)
                    pltpu.sync_copy(r0, acc.at[dst_v.at[c0]], add=True)
                    pltpu.make_async_copy(
                        ht_hbm.at[src_v.at[c0 + 1]], r1, sem1).wait()
                    nxt = jnp.minimum(c0 + 2, nq - 1)
                    pltpu.async_copy(ht_hbm.at[src_v.at[nxt]], r0, sem0)
                    pltpu.sync_copy(r1, acc.at[dst_v.at[c0 + 1]], add=True)
                    return carry

                lax.fori_loop(0, nq // 2, body, 0)
                pltpu.make_async_copy(
                    ht_hbm.at[src_v.at[nq - 1]], r0, sem0).wait()
            plsc.subcore_barrier()
            pltpu.sync_copy(
                acc.at[pl.ds(sid * rpt, rpt)],
                out_hbm.at[pl.ds(s * npad + sid * rpt, rpt)])
            plsc.subcore_barrier()

    return agg_kernel(ht_flat, srcp, dst2, zeros_mat)


def _tc_pre(h, ns_mat, w_l):
    npad, d = h.shape
    n_strips = d // STRIP
    br = 512

    def body(h_ref, ns_ref, w_ref, out_ref):
        x = h_ref[...] * ns_ref[:, :1]
        y = jnp.dot(x, w_ref[...], preferred_element_type=jnp.float32)
        for s in range(n_strips):
            out_ref[s] = y[:, s * STRIP:(s + 1) * STRIP]

    return pl.pallas_call(
        body,
        grid=(npad // br,),
        in_specs=[
            pl.BlockSpec((br, d), lambda i: (i, 0)),
            pl.BlockSpec((br, STRIP), lambda i: (i, 0)),
            pl.BlockSpec((d, d), lambda i: (0, 0)),
        ],
        out_specs=pl.BlockSpec((n_strips, br, STRIP), lambda i: (0, i, 0)),
        out_shape=jax.ShapeDtypeStruct((n_strips, npad, STRIP), jnp.float32),
    )(h, ns_mat, w_l)


def _tc_post(agg, nd_mat, b_l, g_l, bt_l):
    n_strips, npad, strip = agg.shape
    d = n_strips * strip
    br = 512
    inv_sqrt2 = 1.0 / math.sqrt(2.0)

    def body(a_ref, nd_ref, b_ref, g_ref, bt_ref, out_ref):
        x = jnp.concatenate([a_ref[s] for s in range(n_strips)], axis=1)
        x = x * nd_ref[:, :1] + b_ref[...]
        mu = jnp.mean(x, axis=-1, keepdims=True)
        xc = x - mu
        var = jnp.mean(xc * xc, axis=-1, keepdims=True)
        xn = xc * lax.rsqrt(var + 1e-5) * g_ref[...] + bt_ref[...]
        out_ref[...] = 0.5 * xn * (1.0 + lax.erf(xn * inv_sqrt2))

    return pl.pallas_call(
        body,
        grid=(npad // br,),
        in_specs=[
            pl.BlockSpec((n_strips, br, strip), lambda i: (0, i, 0)),
            pl.BlockSpec((br, STRIP), lambda i: (i, 0)),
            pl.BlockSpec((1, d), lambda i: (0, 0)),
            pl.BlockSpec((1, d), lambda i: (0, 0)),
            pl.BlockSpec((1, d), lambda i: (0, 0)),
        ],
        out_specs=pl.BlockSpec((br, d), lambda i: (i, 0)),
        out_shape=jax.ShapeDtypeStruct((npad, d), jnp.float32),
    )(agg, nd_mat, b_l, g_l, bt_l)


def kernel(features, edge_index, W, b, gamma, beta):
    n, d = features.shape
    e = edge_index.shape[1]
    n_layers = W.shape[0]
    n_strips = d // STRIP

    npad = _round_up(n, 2048)
    epad = _round_up(e, NS * CHUNK * QTRS * 2)
    sink = npad - 1

    src = edge_index[0].astype(jnp.int32)
    dst = edge_index[1].astype(jnp.int32)
    src_pad = jnp.concatenate(
        [src, jnp.full((epad - e,), sink, jnp.int32)])
    dst_pad = jnp.concatenate(
        [dst, jnp.full((epad - e,), sink, jnp.int32)])

    srcp = (src_pad[None, :]
            + (jnp.arange(n_strips, dtype=jnp.int32) * npad)[:, None])
    nq = epad // NS // CHUNK // QTRS
    srcp = srcp.reshape(n_strips, NS, QTRS, nq, CHUNK)
    dst2 = dst_pad.reshape(NS, QTRS, nq, CHUNK)

    zeros2 = jnp.zeros((2, npad), jnp.float32)
    zeros_mat = jnp.zeros((npad, STRIP), jnp.float32)

    partials = _sc_degree_partials(src_pad, dst_pad, zeros2)
    norms = _tc_norms(partials)
    ns_mat = jnp.broadcast_to(norms[0][:, None], (npad, STRIP))
    nd_mat = jnp.broadcast_to(norms[1][:, None], (npad, STRIP))

    h = jnp.pad(features, ((0, npad - n), (0, 0)))
    for l in range(n_layers):
        ht = _tc_pre(h, ns_mat, W[l])
        agg = _sc_aggregate(ht.reshape(n_strips * npad, STRIP),
                            srcp, dst2, zeros_mat)
        h = _tc_post(agg.reshape(n_strips, npad, STRIP), nd_mat,
                     b[l].reshape(1, d), gamma[l].reshape(1, d),
                     beta[l].reshape(1, d))
    return h[:n]


# speedup vs baseline: 1.3279x; 1.2030x over previous
import functools
import math

import jax
import jax.numpy as jnp
from jax import lax
from jax.experimental import pallas as pl
from jax.experimental.pallas import tpu as pltpu
from jax.experimental.pallas import tpu_sc as plsc

NC = 2
NS = 16
N_TILES = NC * NS
STRIP = 128
CHUNK = 128


def _round_up(x, m):
    return (x + m - 1) // m * m


def _sc_degree_partials(src_pad, dst_pad, zeros2):
    epad = src_pad.shape[0]
    npad = zeros2.shape[1]
    eper = epad // N_TILES
    mesh = plsc.VectorSubcoreMesh(core_axis_name="c", subcore_axis_name="s")

    @functools.partial(
        pl.kernel,
        out_type=jax.ShapeDtypeStruct((N_TILES, 2, npad), jnp.float32),
        mesh=mesh,
        scratch_types=[
            pltpu.VMEM((eper,), jnp.int32),
            pltpu.VMEM((eper,), jnp.int32),
            pltpu.VMEM((npad,), jnp.float32),
            pltpu.VMEM((npad,), jnp.float32),
        ],
        compiler_params=pltpu.CompilerParams(needs_layout_passes=False),
    )
    def deg_kernel(src_hbm, dst_hbm, z_hbm, out_hbm, src_v, dst_v, hs_v, hd_v):
        cid = lax.axis_index("c")
        sid = lax.axis_index("s")
        wid = cid * NS + sid
        base = wid * eper
        pltpu.sync_copy(z_hbm.at[0], hs_v)
        pltpu.sync_copy(z_hbm.at[1], hd_v)
        pltpu.sync_copy(src_hbm.at[pl.ds(base, eper)], src_v)
        pltpu.sync_copy(dst_hbm.at[pl.ds(base, eper)], dst_v)
        ones = jnp.full((16,), 1.0, jnp.float32)

        def body(g, carry):
            vs = src_v[pl.ds(g * 16, 16)]
            plsc.addupdate_scatter(hs_v, [vs], ones)
            vd = dst_v[pl.ds(g * 16, 16)]
            plsc.addupdate_scatter(hd_v, [vd], ones)
            return carry

        lax.fori_loop(0, eper // 16, body, 0)
        pltpu.sync_copy(hs_v, out_hbm.at[wid, 0])
        pltpu.sync_copy(hd_v, out_hbm.at[wid, 1])

    return deg_kernel(src_pad, dst_pad, zeros2)


def _tc_norms(partials):
    t, two, npad = partials.shape

    def body(p_ref, out_ref):
        x = p_ref[...].reshape(t, two * npad)
        deg = jnp.sum(x, axis=0)
        out_ref[...] = lax.rsqrt(jnp.maximum(deg, 1.0)).reshape(two, npad)

    return pl.pallas_call(
        body,
        out_shape=jax.ShapeDtypeStruct((two, npad), jnp.float32),
    )(partials)


def _sc_aggregate(ht_flat, srcp, dst2, zeros_mat):
    n_strips = srcp.shape[0]
    nch = srcp.shape[2]
    npad = zeros_mat.shape[0]
    rpt = npad // NS
    strips_per_core = n_strips // NC
    mesh = plsc.VectorSubcoreMesh(core_axis_name="c", subcore_axis_name="s")

    @functools.partial(
        pl.kernel,
        out_type=jax.ShapeDtypeStruct((n_strips * npad, STRIP), jnp.float32),
        mesh=mesh,
        scratch_types=[
            pltpu.VMEM((nch, CHUNK), jnp.int32),
            pltpu.VMEM((nch, CHUNK), jnp.int32),
            pltpu.VMEM((CHUNK, STRIP), jnp.float32),
            pltpu.VMEM_SHARED((npad, STRIP), jnp.float32),
        ],
    )
    def agg_kernel(ht_hbm, src_hbm, dst_hbm, z_hbm, out_hbm,
                   src_v, dst_v, rows_v, acc):
        cid = lax.axis_index("c")
        sid = lax.axis_index("s")
        pltpu.sync_copy(dst_hbm.at[sid], dst_v)
        for j in range(strips_per_core):
            s = cid * strips_per_core + j
            pltpu.sync_copy(z_hbm.at[pl.ds(sid * rpt, rpt)],
                            acc.at[pl.ds(sid * rpt, rpt)])
            pltpu.sync_copy(src_hbm.at[s, sid], src_v)
            plsc.subcore_barrier()

            def body(i, carry):
                pltpu.sync_copy(ht_hbm.at[src_v.at[i]], rows_v)
                pltpu.sync_copy(rows_v, acc.at[dst_v.at[i]], add=True)
                return carry

            lax.fori_loop(0, nch, body, 0)
            plsc.subcore_barrier()
            pltpu.sync_copy(
                acc.at[pl.ds(sid * rpt, rpt)],
                out_hbm.at[pl.ds(s * npad + sid * rpt, rpt)])
            plsc.subcore_barrier()

    return agg_kernel(ht_flat, srcp, dst2, zeros_mat)


def _tc_pre(h, ns_mat, w_l):
    npad, d = h.shape
    n_strips = d // STRIP
    br = 512

    def body(h_ref, ns_ref, w_ref, out_ref):
        x = h_ref[...] * ns_ref[:, :1]
        y = jnp.dot(x, w_ref[...], preferred_element_type=jnp.float32)
        for s in range(n_strips):
            out_ref[s] = y[:, s * STRIP:(s + 1) * STRIP]

    return pl.pallas_call(
        body,
        grid=(npad // br,),
        in_specs=[
            pl.BlockSpec((br, d), lambda i: (i, 0)),
            pl.BlockSpec((br, STRIP), lambda i: (i, 0)),
            pl.BlockSpec((d, d), lambda i: (0, 0)),
        ],
        out_specs=pl.BlockSpec((n_strips, br, STRIP), lambda i: (0, i, 0)),
        out_shape=jax.ShapeDtypeStruct((n_strips, npad, STRIP), jnp.float32),
    )(h, ns_mat, w_l)


def _tc_post(agg, nd_mat, b_l, g_l, bt_l):
    n_strips, npad, strip = agg.shape
    d = n_strips * strip
    br = 512
    inv_sqrt2 = 1.0 / math.sqrt(2.0)

    def body(a_ref, nd_ref, b_ref, g_ref, bt_ref, out_ref):
        x = jnp.concatenate([a_ref[s] for s in range(n_strips)], axis=1)
        x = x * nd_ref[:, :1] + b_ref[...]
        mu = jnp.mean(x, axis=-1, keepdims=True)
        xc = x - mu
        var = jnp.mean(xc * xc, axis=-1, keepdims=True)
        xn = xc * lax.rsqrt(var + 1e-5) * g_ref[...] + bt_ref[...]
        out_ref[...] = 0.5 * xn * (1.0 + lax.erf(xn * inv_sqrt2))

    return pl.pallas_call(
        body,
        grid=(npad // br,),
        in_specs=[
            pl.BlockSpec((n_strips, br, strip), lambda i: (0, i, 0)),
            pl.BlockSpec((br, STRIP), lambda i: (i, 0)),
            pl.BlockSpec((1, d), lambda i: (0, 0)),
            pl.BlockSpec((1, d), lambda i: (0, 0)),
            pl.BlockSpec((1, d), lambda i: (0, 0)),
        ],
        out_specs=pl.BlockSpec((br, d), lambda i: (i, 0)),
        out_shape=jax.ShapeDtypeStruct((npad, d), jnp.float32),
    )(agg, nd_mat, b_l, g_l, bt_l)


def kernel(features, edge_index, W, b, gamma, beta):
    n, d = features.shape
    e = edge_index.shape[1]
    n_layers = W.shape[0]
    n_strips = d // STRIP

    npad = _round_up(n, 2048)
    epad = _round_up(e, NS * CHUNK)
    sink = npad - 1

    src = edge_index[0].astype(jnp.int32)
    dst = edge_index[1].astype(jnp.int32)
    order = jnp.argsort(dst)
    src_pad = jnp.concatenate(
        [src[order], jnp.full((epad - e,), sink, jnp.int32)])
    dst_pad = jnp.concatenate(
        [dst[order], jnp.full((epad - e,), sink, jnp.int32)])

    srcp = (src_pad[None, :]
            + (jnp.arange(n_strips, dtype=jnp.int32) * npad)[:, None])
    nch = epad // NS // CHUNK
    srcp = srcp.reshape(n_strips, NS, nch, CHUNK)
    dst2 = dst_pad.reshape(NS, nch, CHUNK)

    zeros2 = jnp.zeros((2, npad), jnp.float32)
    zeros_mat = jnp.zeros((npad, STRIP), jnp.float32)

    partials = _sc_degree_partials(src_pad, dst_pad, zeros2)
    norms = _tc_norms(partials)
    ns_mat = jnp.broadcast_to(norms[0][:, None], (npad, STRIP))
    nd_mat = jnp.broadcast_to(norms[1][:, None], (npad, STRIP))

    h = jnp.pad(features, ((0, npad - n), (0, 0)))
    for l in range(n_layers):
        ht = _tc_pre(h, ns_mat, W[l])
        agg = _sc_aggregate(ht.reshape(n_strips * npad, STRIP),
                            srcp, dst2, zeros_mat)
        h = _tc_post(agg.reshape(n_strips, npad, STRIP), nd_mat,
                     b[l].reshape(1, d), gamma[l].reshape(1, d),
                     beta[l].reshape(1, d))
    return h[:n]


# speedup vs baseline: 1.4665x; 1.1044x over previous
import functools
import math

import jax
import jax.numpy as jnp
from jax import lax
from jax.experimental import pallas as pl
from jax.experimental.pallas import tpu as pltpu
from jax.experimental.pallas import tpu_sc as plsc

NC = 2
NS = 16
N_TILES = NC * NS
STRIP = 128
CHUNK = 128


def _round_up(x, m):
    return (x + m - 1) // m * m


def _sc_degree_partials(src_pad, dst_pad, zeros2):
    epad = src_pad.shape[0]
    npad = zeros2.shape[1]
    eper = epad // N_TILES
    mesh = plsc.VectorSubcoreMesh(core_axis_name="c", subcore_axis_name="s")

    @functools.partial(
        pl.kernel,
        out_type=jax.ShapeDtypeStruct((N_TILES, 2, npad), jnp.float32),
        mesh=mesh,
        scratch_types=[
            pltpu.VMEM((eper,), jnp.int32),
            pltpu.VMEM((eper,), jnp.int32),
            pltpu.VMEM((npad,), jnp.float32),
            pltpu.VMEM((npad,), jnp.float32),
        ],
        compiler_params=pltpu.CompilerParams(needs_layout_passes=False),
    )
    def deg_kernel(src_hbm, dst_hbm, z_hbm, out_hbm, src_v, dst_v, hs_v, hd_v):
        cid = lax.axis_index("c")
        sid = lax.axis_index("s")
        wid = cid * NS + sid
        base = wid * eper
        pltpu.sync_copy(z_hbm.at[0], hs_v)
        pltpu.sync_copy(z_hbm.at[1], hd_v)
        pltpu.sync_copy(src_hbm.at[pl.ds(base, eper)], src_v)
        pltpu.sync_copy(dst_hbm.at[pl.ds(base, eper)], dst_v)
        ones = jnp.full((16,), 1.0, jnp.float32)

        def body(g, carry):
            vs = src_v[pl.ds(g * 16, 16)]
            plsc.addupdate_scatter(hs_v, [vs], ones)
            vd = dst_v[pl.ds(g * 16, 16)]
            plsc.addupdate_scatter(hd_v, [vd], ones)
            return carry

        lax.fori_loop(0, eper // 16, body, 0)
        pltpu.sync_copy(hs_v, out_hbm.at[wid, 0])
        pltpu.sync_copy(hd_v, out_hbm.at[wid, 1])

    return deg_kernel(src_pad, dst_pad, zeros2)


def _tc_norms(partials):
    t, two, npad = partials.shape

    def body(p_ref, out_ref):
        x = p_ref[...].reshape(t, two * npad)
        deg = jnp.sum(x, axis=0)
        out_ref[...] = lax.rsqrt(jnp.maximum(deg, 1.0)).reshape(two, npad)

    return pl.pallas_call(
        body,
        out_shape=jax.ShapeDtypeStruct((two, npad), jnp.float32),
    )(partials)


def _sc_aggregate(ht_flat, srcp, dst2, zeros_mat):
    n_strips = srcp.shape[0]
    nch = srcp.shape[2]
    npad = zeros_mat.shape[0]
    rpt = npad // NS
    strips_per_core = n_strips // NC
    mesh = plsc.VectorSubcoreMesh(core_axis_name="c", subcore_axis_name="s")

    @functools.partial(
        pl.kernel,
        out_type=jax.ShapeDtypeStruct((n_strips * npad, STRIP), jnp.float32),
        mesh=mesh,
        scratch_types=[
            pltpu.VMEM((nch, CHUNK), jnp.int32),
            pltpu.VMEM((nch, CHUNK), jnp.int32),
            pltpu.VMEM((CHUNK, STRIP), jnp.float32),
            pltpu.VMEM_SHARED((npad, STRIP), jnp.float32),
        ],
    )
    def agg_kernel(ht_hbm, src_hbm, dst_hbm, z_hbm, out_hbm,
                   src_v, dst_v, rows_v, acc):
        cid = lax.axis_index("c")
        sid = lax.axis_index("s")
        pltpu.sync_copy(dst_hbm.at[sid], dst_v)
        for j in range(strips_per_core):
            s = cid * strips_per_core + j
            pltpu.sync_copy(z_hbm.at[pl.ds(sid * rpt, rpt)],
                            acc.at[pl.ds(sid * rpt, rpt)])
            pltpu.sync_copy(src_hbm.at[s, sid], src_v)
            plsc.subcore_barrier()

            def body(i, carry):
                pltpu.sync_copy(ht_hbm.at[src_v.at[i]], rows_v)
                pltpu.sync_copy(rows_v, acc.at[dst_v.at[i]], add=True)
                return carry

            lax.fori_loop(0, nch, body, 0)
            plsc.subcore_barrier()
            pltpu.sync_copy(
                acc.at[pl.ds(sid * rpt, rpt)],
                out_hbm.at[pl.ds(s * npad + sid * rpt, rpt)])
            plsc.subcore_barrier()

    return agg_kernel(ht_flat, srcp, dst2, zeros_mat)


def _tc_pre(h, ns_mat, w_l):
    npad, d = h.shape
    n_strips = d // STRIP
    br = 512

    def body(h_ref, ns_ref, w_ref, out_ref):
        x = h_ref[...] * ns_ref[:, :1]
        y = jnp.dot(x, w_ref[...], preferred_element_type=jnp.float32)
        for s in range(n_strips):
            out_ref[s] = y[:, s * STRIP:(s + 1) * STRIP]

    return pl.pallas_call(
        body,
        grid=(npad // br,),
        in_specs=[
            pl.BlockSpec((br, d), lambda i: (i, 0)),
            pl.BlockSpec((br, STRIP), lambda i: (i, 0)),
            pl.BlockSpec((d, d), lambda i: (0, 0)),
        ],
        out_specs=pl.BlockSpec((n_strips, br, STRIP), lambda i: (0, i, 0)),
        out_shape=jax.ShapeDtypeStruct((n_strips, npad, STRIP), jnp.float32),
    )(h, ns_mat, w_l)


def _tc_fused(agg, nd_mat, b_l, g_l, bt_l, ns_mat, w_next):
    n_strips, npad, strip = agg.shape
    d = n_strips * strip
    br = 512
    inv_sqrt2 = 1.0 / math.sqrt(2.0)

    def body(a_ref, nd_ref, b_ref, g_ref, bt_ref, ns_ref, w_ref, out_ref):
        x = jnp.concatenate([a_ref[s] for s in range(n_strips)], axis=1)
        x = x * nd_ref[:, :1] + b_ref[...]
        mu = jnp.mean(x, axis=-1, keepdims=True)
        xc = x - mu
        var = jnp.mean(xc * xc, axis=-1, keepdims=True)
        xn = xc * lax.rsqrt(var + 1e-5) * g_ref[...] + bt_ref[...]
        h = 0.5 * xn * (1.0 + lax.erf(xn * inv_sqrt2))
        y = jnp.dot(h * ns_ref[:, :1], w_ref[...],
                    preferred_element_type=jnp.float32)
        for s in range(n_strips):
            out_ref[s] = y[:, s * STRIP:(s + 1) * STRIP]

    return pl.pallas_call(
        body,
        grid=(npad // br,),
        in_specs=[
            pl.BlockSpec((n_strips, br, strip), lambda i: (0, i, 0)),
            pl.BlockSpec((br, STRIP), lambda i: (i, 0)),
            pl.BlockSpec((1, d), lambda i: (0, 0)),
            pl.BlockSpec((1, d), lambda i: (0, 0)),
            pl.BlockSpec((1, d), lambda i: (0, 0)),
            pl.BlockSpec((br, STRIP), lambda i: (i, 0)),
            pl.BlockSpec((d, d), lambda i: (0, 0)),
        ],
        out_specs=pl.BlockSpec((n_strips, br, STRIP), lambda i: (0, i, 0)),
        out_shape=jax.ShapeDtypeStruct((n_strips, npad, STRIP), jnp.float32),
    )(agg, nd_mat, b_l, g_l, bt_l, ns_mat, w_next)


def _tc_post(agg, nd_mat, b_l, g_l, bt_l):
    n_strips, npad, strip = agg.shape
    d = n_strips * strip
    br = 512
    inv_sqrt2 = 1.0 / math.sqrt(2.0)

    def body(a_ref, nd_ref, b_ref, g_ref, bt_ref, out_ref):
        x = jnp.concatenate([a_ref[s] for s in range(n_strips)], axis=1)
        x = x * nd_ref[:, :1] + b_ref[...]
        mu = jnp.mean(x, axis=-1, keepdims=True)
        xc = x - mu
        var = jnp.mean(xc * xc, axis=-1, keepdims=True)
        xn = xc * lax.rsqrt(var + 1e-5) * g_ref[...] + bt_ref[...]
        out_ref[...] = 0.5 * xn * (1.0 + lax.erf(xn * inv_sqrt2))

    return pl.pallas_call(
        body,
        grid=(npad // br,),
        in_specs=[
            pl.BlockSpec((n_strips, br, strip), lambda i: (0, i, 0)),
            pl.BlockSpec((br, STRIP), lambda i: (i, 0)),
            pl.BlockSpec((1, d), lambda i: (0, 0)),
            pl.BlockSpec((1, d), lambda i: (0, 0)),
            pl.BlockSpec((1, d), lambda i: (0, 0)),
        ],
        out_specs=pl.BlockSpec((br, d), lambda i: (i, 0)),
        out_shape=jax.ShapeDtypeStruct((npad, d), jnp.float32),
    )(agg, nd_mat, b_l, g_l, bt_l)


def kernel(features, edge_index, W, b, gamma, beta):
    n, d = features.shape
    e = edge_index.shape[1]
    n_layers = W.shape[0]
    n_strips = d // STRIP

    npad = _round_up(n, 2048)
    epad = _round_up(e, NS * CHUNK)
    sink = npad - 1

    src = edge_index[0].astype(jnp.int32)
    dst = edge_index[1].astype(jnp.int32)
    src_pad = jnp.concatenate(
        [src, jnp.full((epad - e,), sink, jnp.int32)])
    dst_pad = jnp.concatenate(
        [dst, jnp.full((epad - e,), sink, jnp.int32)])

    srcp = (src_pad[None, :]
            + (jnp.arange(n_strips, dtype=jnp.int32) * npad)[:, None])
    nch = epad // NS // CHUNK
    srcp = srcp.reshape(n_strips, NS, nch, CHUNK)
    dst2 = dst_pad.reshape(NS, nch, CHUNK)

    zeros2 = jnp.zeros((2, npad), jnp.float32)
    zeros_mat = jnp.zeros((npad, STRIP), jnp.float32)

    partials = _sc_degree_partials(src_pad, dst_pad, zeros2)
    norms = _tc_norms(partials)
    ns_mat = jnp.broadcast_to(norms[0][:, None], (npad, STRIP))
    nd_mat = jnp.broadcast_to(norms[1][:, None], (npad, STRIP))

    h0 = jnp.pad(features, ((0, npad - n), (0, 0)))
    ht = _tc_pre(h0, ns_mat, W[0])
    for l in range(n_layers):
        agg = _sc_aggregate(ht.reshape(n_strips * npad, STRIP),
                            srcp, dst2, zeros_mat)
        agg = agg.reshape(n_strips, npad, STRIP)
        if l + 1 < n_layers:
            ht = _tc_fused(agg, nd_mat, b[l].reshape(1, d),
                           gamma[l].reshape(1, d), beta[l].reshape(1, d),
                           ns_mat, W[l + 1])
        else:
            h = _tc_post(agg, nd_mat, b[l].reshape(1, d),
                         gamma[l].reshape(1, d), beta[l].reshape(1, d))
    return h[:n]
